# Initial kernel scaffold; baseline (speedup 1.0000x reference)
import jax
import jax.numpy as jnp
from jax.experimental import pallas as pl


def kernel(x, W):
    raise NotImplementedError("write your pallas kernel here")


# speedup vs baseline: 1.0529x; 1.0529x over previous
import jax
import jax.numpy as jnp
from jax.experimental import pallas as pl
from jax.experimental.pallas import tpu as pltpu

_BLOCK_M = 512
_N_CODES = 1024
_DIM = 64


def _vq_body(xt_ref, w_ref, o_ref):
    w = w_ref[...]
    xt = xt_ref[...]
    dots = jax.lax.dot_general(
        w, xt, (((1,), (0,)), ((), ())),
        preferred_element_type=jnp.float32)
    wsq = jnp.sum(w * w, axis=1, keepdims=True)
    xsq = jnp.sum(xt * xt, axis=0, keepdims=True)
    d = (xsq + wsq) - 2.0 * dots
    m = jnp.min(d, axis=0, keepdims=True)
    ids = jax.lax.broadcasted_iota(jnp.int32, d.shape, 0)
    cand = jnp.where(d == m, ids, _N_CODES)
    idx = jnp.min(cand, axis=0, keepdims=True)
    o_ref[...] = idx[None]


def kernel(x, W):
    n = x.shape[0]
    grid = n // _BLOCK_M
    xt = x.T
    out = pl.pallas_call(
        _vq_body,
        grid=(grid,),
        in_specs=[
            pl.BlockSpec((_DIM, _BLOCK_M), lambda i: (0, i)),
            pl.BlockSpec((_N_CODES, _DIM), lambda i: (0, 0)),
        ],
        out_specs=pl.BlockSpec((1, 1, _BLOCK_M), lambda i: (i, 0, 0)),
        out_shape=jax.ShapeDtypeStruct((grid, 1, _BLOCK_M), jnp.int32),
        compiler_params=pltpu.CompilerParams(
            dimension_semantics=("arbitrary",)),
    )(xt, W)
    return out.reshape(n)
